# baseline (speedup 1.0000x reference)
import jax
import jax.numpy as jnp
import numpy as np
from jax.experimental import pallas as pl
from jax.experimental.pallas import tpu as pltpu

FS = 2048.0
K = 101
FC_MIN, FC_MAX = 300.0, 550.0
EPS = 1e-5
BLOCK_SPEC = [(2, 8, 1), (8, 8, 1), (8, 8, 1), (8, 16, 4), (16, 16, 1), (16, 16, 1),
              (16, 16, 1), (16, 32, 4), (32, 32, 1), (32, 32, 1), (32, 32, 1),
              (32, 32, 1), (32, 32, 1), (32, 64, 4), (64, 64, 1), (64, 64, 1)]


def _conv1d(x, w, b, stride=1, pad=0):
    y = jax.lax.conv_general_dilated(x, w, (stride,), [(pad, pad)],
                                     dimension_numbers=("NCH", "OIH", "NCH"))
    return y + b[None, :, None]


def _bn(x, p):
    if x.ndim == 3:
        g, b, m, v = (p["gamma"][None, :, None], p["beta"][None, :, None],
                      p["mean"][None, :, None], p["var"][None, :, None])
    else:
        g, b, m, v = p["gamma"][None, :], p["beta"][None, :], p["mean"][None, :], p["var"][None, :]
    return g * (x - m) * jax.lax.rsqrt(v + EPS) + b


def _resblock(x, p, stride):
    h = _conv1d(x, p["conv1_w"], p["conv1_b"], 1, 1)
    h = jax.nn.relu(_bn(h, p["bn1"]))
    h = _conv1d(h, p["conv2_w"], p["conv2_b"], stride, 1)
    h = _bn(h, p["bn2"])
    sc = _conv1d(x, p["xt_w"], p["xt_b"], stride, 0) if "xt_w" in p else x
    return jax.nn.relu(h + sc)


def _predict_taps(x, params):
    h = x
    for p, (_, _, s) in zip(params["blocks"], BLOCK_SPEC):
        h = _resblock(h, p, s)
    h = h.reshape(h.shape[0], -1)
    h = jax.nn.relu(_bn(h @ params["fc1_w"].T + params["fc1_b"], params["bn_fc"]))
    fc_norm = jax.nn.sigmoid(h @ params["fc2_w"].T + params["fc2_b"])
    fc_hz = FC_MIN + fc_norm * (FC_MAX - FC_MIN)
    t = jnp.arange(-(K // 2), K // 2 + 1, dtype=jnp.float32)
    fc = fc_hz / FS
    kern = 2.0 * fc * jnp.sinc(2.0 * fc * t[None, :])
    win = 0.54 - 0.46 * jnp.cos(2.0 * jnp.pi * jnp.arange(K, dtype=jnp.float32) / K)
    kern = kern * win
    kern = kern / jnp.sum(kern, axis=-1, keepdims=True)
    return kern


_NB = 128
_HALF = K // 2


def _filt_body(x_ref, k_ref, o_ref):
    xt = x_ref[...]
    kt = k_ref[...]
    nb, T = xt.shape
    zpad = jnp.zeros((nb, _HALF), dtype=xt.dtype)
    xp = jnp.concatenate([zpad, xt, zpad], axis=1)
    CW = 512
    for c0 in range(0, T, CW):
        win = xp[:, c0:c0 + CW + K - 1]
        acc = kt[:, 0:1] * win[:, 0:CW]
        for k in range(1, K):
            acc += kt[:, k:k + 1] * win[:, k:k + CW]
        o_ref[:, c0:c0 + CW] = acc


def _apply_filter(x, kern):
    B, C, T = x.shape
    rows = B * C
    x2 = x.reshape(rows, T)
    kpad = jnp.pad(kern, ((0, 0), (0, 128 - K)))
    kr = jnp.repeat(kpad, C, axis=0)
    grid = (rows // _NB,)
    y = pl.pallas_call(
        _filt_body,
        out_shape=jax.ShapeDtypeStruct((rows, T), x.dtype),
        grid=grid,
        in_specs=[pl.BlockSpec((_NB, T), lambda i: (i, 0)),
                  pl.BlockSpec((_NB, 128), lambda i: (i, 0))],
        out_specs=pl.BlockSpec((_NB, T), lambda i: (i, 0)),
        compiler_params=pltpu.CompilerParams(
            dimension_semantics=("arbitrary",)),
        name="adaptive_fir",
    )(x2, kr)
    return y.reshape(B, C, T)


def kernel(x, params):
    kern = _predict_taps(x, params)
    return _apply_filter(x, kern)


# speedup vs baseline: 1.5180x; 1.5180x over previous
import jax
import jax.numpy as jnp
import numpy as np
from jax.experimental import pallas as pl
from jax.experimental.pallas import tpu as pltpu

FS = 2048.0
K = 101
FC_MIN, FC_MAX = 300.0, 550.0
EPS = 1e-5
BLOCK_SPEC = [(2, 8, 1), (8, 8, 1), (8, 8, 1), (8, 16, 4), (16, 16, 1), (16, 16, 1),
              (16, 16, 1), (16, 32, 4), (32, 32, 1), (32, 32, 1), (32, 32, 1),
              (32, 32, 1), (32, 32, 1), (32, 64, 4), (64, 64, 1), (64, 64, 1)]


def _conv1d(x, w, b, stride=1, pad=0):
    y = jax.lax.conv_general_dilated(x, w, (stride,), [(pad, pad)],
                                     dimension_numbers=("NCH", "OIH", "NCH"))
    return y + b[None, :, None]


def _bn(x, p):
    if x.ndim == 3:
        g, b, m, v = (p["gamma"][None, :, None], p["beta"][None, :, None],
                      p["mean"][None, :, None], p["var"][None, :, None])
    else:
        g, b, m, v = p["gamma"][None, :], p["beta"][None, :], p["mean"][None, :], p["var"][None, :]
    return g * (x - m) * jax.lax.rsqrt(v + EPS) + b


def _resblock(x, p, stride):
    h = _conv1d(x, p["conv1_w"], p["conv1_b"], 1, 1)
    h = jax.nn.relu(_bn(h, p["bn1"]))
    h = _conv1d(h, p["conv2_w"], p["conv2_b"], stride, 1)
    h = _bn(h, p["bn2"])
    sc = _conv1d(x, p["xt_w"], p["xt_b"], stride, 0) if "xt_w" in p else x
    return jax.nn.relu(h + sc)


def _predict_fc(x, params):
    h = x
    for p, (_, _, s) in zip(params["blocks"], BLOCK_SPEC):
        h = _resblock(h, p, s)
    h = h.reshape(h.shape[0], -1)
    h = jax.nn.relu(_bn(h @ params["fc1_w"].T + params["fc1_b"], params["bn_fc"]))
    fc_norm = jax.nn.sigmoid(h @ params["fc2_w"].T + params["fc2_b"])
    return FC_MIN + fc_norm * (FC_MAX - FC_MIN)


_NR = 128
_HALF = K // 2
_T = 2048
_PT = 2152


def _filt_body(fc_ref, x_ref, o_ref):
    fc = fc_ref[0, :] / FS
    kidx = jax.lax.broadcasted_iota(jnp.int32, (104, _NR), 0).astype(jnp.float32)
    t = kidx - float(_HALF)
    z = 2.0 * fc[None, :] * t
    pz = np.float32(np.pi) * z
    sinc = jnp.where(t == 0.0, 1.0, jnp.sin(pz) / jnp.where(pz == 0.0, 1.0, pz))
    win = 0.54 - 0.46 * jnp.cos(np.float32(2.0 * np.pi / K) * kidx)
    valid = kidx < float(K)
    kern = jnp.where(valid, 2.0 * fc[None, :] * sinc * win, 0.0)
    kern = kern / jnp.sum(kern, axis=0, keepdims=True)
    xt = x_ref[...]
    acc = jnp.zeros((_T, _NR), dtype=jnp.float32)
    for s in range(8):
        if s == 0:
            xs = xt
        else:
            xs = jnp.concatenate(
                [xt[s:, :], jnp.zeros((s, _NR), dtype=jnp.float32)], axis=0)
        for q in range(13):
            k = 8 * q + s
            if k >= K:
                continue
            acc = acc + kern[k, :][None, :] * xs[8 * q:8 * q + _T, :]
    o_ref[...] = acc


def _apply_filter(x, fc_hz):
    B, C, T = x.shape
    rows = B * C
    xt = jnp.pad(x.reshape(rows, T).T, ((_HALF, _PT - _T - _HALF), (0, 0)))
    fcr = jnp.broadcast_to(fc_hz[:, None, :], (B, C, 1)).reshape(1, rows)
    fcr = jnp.broadcast_to(fcr, (8, rows))
    grid = (rows // _NR,)
    y = pl.pallas_call(
        _filt_body,
        out_shape=jax.ShapeDtypeStruct((T, rows), x.dtype),
        grid=grid,
        in_specs=[pl.BlockSpec((8, _NR), lambda i: (0, i)),
                  pl.BlockSpec((_PT, _NR), lambda i: (0, i))],
        out_specs=pl.BlockSpec((_T, _NR), lambda i: (0, i)),
        compiler_params=pltpu.CompilerParams(
            dimension_semantics=("arbitrary",)),
        name="adaptive_fir",
    )(fcr, xt)
    return y.T.reshape(B, C, T)


def kernel(x, params):
    fc_hz = _predict_fc(x, params)
    return _apply_filter(x, fc_hz)


# speedup vs baseline: 1.5450x; 1.0178x over previous
import jax
import jax.numpy as jnp
import numpy as np
from jax.experimental import pallas as pl
from jax.experimental.pallas import tpu as pltpu

FS = 2048.0
K = 101
FC_MIN, FC_MAX = 300.0, 550.0
EPS = 1e-5
BLOCK_SPEC = [(2, 8, 1), (8, 8, 1), (8, 8, 1), (8, 16, 4), (16, 16, 1), (16, 16, 1),
              (16, 16, 1), (16, 32, 4), (32, 32, 1), (32, 32, 1), (32, 32, 1),
              (32, 32, 1), (32, 32, 1), (32, 64, 4), (64, 64, 1), (64, 64, 1)]


def _conv1d(x, w, b, stride=1, pad=0):
    y = jax.lax.conv_general_dilated(x, w, (stride,), [(pad, pad)],
                                     dimension_numbers=("NCH", "OIH", "NCH"))
    return y + b[None, :, None]


def _bn(x, p):
    if x.ndim == 3:
        g, b, m, v = (p["gamma"][None, :, None], p["beta"][None, :, None],
                      p["mean"][None, :, None], p["var"][None, :, None])
    else:
        g, b, m, v = p["gamma"][None, :], p["beta"][None, :], p["mean"][None, :], p["var"][None, :]
    return g * (x - m) * jax.lax.rsqrt(v + EPS) + b


def _resblock(x, p, stride):
    h = _conv1d(x, p["conv1_w"], p["conv1_b"], 1, 1)
    h = jax.nn.relu(_bn(h, p["bn1"]))
    h = _conv1d(h, p["conv2_w"], p["conv2_b"], stride, 1)
    h = _bn(h, p["bn2"])
    sc = _conv1d(x, p["xt_w"], p["xt_b"], stride, 0) if "xt_w" in p else x
    return jax.nn.relu(h + sc)


def _predict_fc(x, params):
    h = x
    for p, (_, _, s) in zip(params["blocks"], BLOCK_SPEC):
        h = _resblock(h, p, s)
    h = h.reshape(h.shape[0], -1)
    h = jax.nn.relu(_bn(h @ params["fc1_w"].T + params["fc1_b"], params["bn_fc"]))
    fc_norm = jax.nn.sigmoid(h @ params["fc2_w"].T + params["fc2_b"])
    return FC_MIN + fc_norm * (FC_MAX - FC_MIN)


_NR = 128
_HALF = K // 2
_T = 2048
_PT = 2152


def _filt_body(fc_ref, x_ref, o_ref):
    fc = fc_ref[0, :] / FS
    kidx = jax.lax.broadcasted_iota(jnp.int32, (104, _NR), 0).astype(jnp.float32)
    t = kidx - float(_HALF)
    z = 2.0 * fc[None, :] * t
    pz = np.float32(np.pi) * z
    sinc = jnp.where(t == 0.0, 1.0, jnp.sin(pz) / jnp.where(pz == 0.0, 1.0, pz))
    win = 0.54 - 0.46 * jnp.cos(np.float32(2.0 * np.pi / K) * kidx)
    valid = kidx < float(K)
    kern = jnp.where(valid, 2.0 * fc[None, :] * sinc * win, 0.0)
    kern = kern / jnp.sum(kern, axis=0, keepdims=True)
    xt = x_ref[...]
    xs_list = [xt]
    for s in range(1, 8):
        xs_list.append(jnp.concatenate(
            [xt[s:, :], jnp.zeros((s, _NR), dtype=jnp.float32)], axis=0))
    CT = 512
    for t0 in range(0, _T, CT):
        accs = [jnp.zeros((CT, _NR), dtype=jnp.float32) for _ in range(4)]
        for k in range(K):
            q, s = divmod(k, 8)
            accs[k % 4] = accs[k % 4] + kern[k, :][None, :] * \
                xs_list[s][t0 + 8 * q:t0 + 8 * q + CT, :]
        o_ref[t0:t0 + CT, :] = (accs[0] + accs[1]) + (accs[2] + accs[3])


def _apply_filter(x, fc_hz):
    B, C, T = x.shape
    rows = B * C
    xt = jnp.pad(x.reshape(rows, T).T, ((_HALF, _PT - _T - _HALF), (0, 0)))
    fcr = jnp.broadcast_to(fc_hz[:, None, :], (B, C, 1)).reshape(1, rows)
    fcr = jnp.broadcast_to(fcr, (8, rows))
    grid = (rows // _NR,)
    y = pl.pallas_call(
        _filt_body,
        out_shape=jax.ShapeDtypeStruct((T, rows), x.dtype),
        grid=grid,
        in_specs=[pl.BlockSpec((8, _NR), lambda i: (0, i)),
                  pl.BlockSpec((_PT, _NR), lambda i: (0, i))],
        out_specs=pl.BlockSpec((_T, _NR), lambda i: (0, i)),
        compiler_params=pltpu.CompilerParams(
            dimension_semantics=("arbitrary",)),
        name="adaptive_fir",
    )(fcr, xt)
    return y.T.reshape(B, C, T)


def kernel(x, params):
    fc_hz = _predict_fc(x, params)
    return _apply_filter(x, fc_hz)
